# baseline (device time: 24492 ns/iter reference)
import jax
import jax.numpy as jnp
from jax import lax
from jax.experimental import pallas as pl
from jax.experimental.pallas import tpu as pltpu

N_DEV = 16


def kernel(x, dy, gamma):
    m, d = x.shape

    def body(x_ref, dy_ref, gamma_ref, out_ref, send_buf, recv_buf,
             send_sems, recv_sems):
        my_pos = lax.axis_index("i")

        barrier_sem = pltpu.get_barrier_semaphore()
        for dd in range(1, N_DEV):
            peer = lax.rem(my_pos + dd, N_DEV)
            pl.semaphore_signal(
                barrier_sem, inc=1,
                device_id=(peer,), device_id_type=pl.DeviceIdType.MESH,
            )

        xv = x_ref[:, :]
        dyv = dy_ref[:, :]
        ones_d = jnp.ones((d, 1), jnp.float32)
        s1 = jax.lax.dot(xv, ones_d, precision=jax.lax.Precision.HIGHEST)
        s2 = jax.lax.dot(xv * xv, ones_d, precision=jax.lax.Precision.HIGHEST)
        mu = s1 / d
        var = s2 / d - mu * mu
        rstd = lax.rsqrt(var + 1e-5)
        t1 = jax.lax.dot(rstd.T, dyv * xv,
                         precision=jax.lax.Precision.HIGHEST)
        w = jnp.concatenate([rstd * mu, jnp.ones((m, 1), jnp.float32)], axis=1)
        t23 = jax.lax.dot(w.T, dyv,
                          precision=jax.lax.Precision.HIGHEST)
        dgamma = t1 - t23[0:1]
        dbeta = t23[1:2]
        partial = jnp.concatenate([dgamma, dbeta], axis=0)
        send_buf[:, :] = partial

        pl.semaphore_wait(barrier_sem, N_DEV - 1)

        rdmas = []
        for dd in range(1, N_DEV):
            peer = lax.rem(my_pos + dd, N_DEV)
            rdma = pltpu.make_async_remote_copy(
                src_ref=send_buf,
                dst_ref=recv_buf.at[dd - 1],
                send_sem=send_sems.at[dd - 1],
                recv_sem=recv_sems.at[dd - 1],
                device_id=(peer,),
                device_id_type=pl.DeviceIdType.MESH,
            )
            rdma.start()
            rdmas.append(rdma)

        for rdma in rdmas:
            rdma.wait_recv()
        out_ref[:, :] = partial + jnp.sum(recv_buf[:, :, :], axis=0)

        for rdma in rdmas:
            rdma.wait_send()

    return pl.pallas_call(
        body,
        out_shape=jax.ShapeDtypeStruct((2, d), jnp.float32),
        in_specs=[
            pl.BlockSpec(memory_space=pltpu.VMEM),
            pl.BlockSpec(memory_space=pltpu.VMEM),
            pl.BlockSpec(memory_space=pltpu.VMEM),
        ],
        out_specs=pl.BlockSpec(memory_space=pltpu.VMEM),
        scratch_shapes=[
            pltpu.VMEM((2, d), jnp.float32),
            pltpu.VMEM((N_DEV - 1, 2, d), jnp.float32),
            pltpu.SemaphoreType.DMA((N_DEV - 1,)),
            pltpu.SemaphoreType.DMA((N_DEV - 1,)),
        ],
        compiler_params=pltpu.CompilerParams(collective_id=0),
    )(x, dy, gamma)


# device time: 14898 ns/iter; 1.6440x vs baseline; 1.6440x over previous
import jax
import jax.numpy as jnp
from jax import lax
from jax.experimental import pallas as pl
from jax.experimental.pallas import tpu as pltpu

N_DEV = 16


def kernel(x, dy, gamma):
    m, d = x.shape

    def body(x_ref, dy_ref, gamma_ref, out_ref, send_buf, recv_buf,
             send_sems, recv_sems):
        my_pos = lax.axis_index("i")

        barrier_sem = pltpu.get_barrier_semaphore()
        for dd in range(1, N_DEV):
            peer = lax.rem(my_pos + dd, N_DEV)
            pl.semaphore_signal(
                barrier_sem, inc=1,
                device_id=(peer,), device_id_type=pl.DeviceIdType.MESH,
            )

        xv = x_ref[:, :]
        dyv = dy_ref[:, :]
        mu = jnp.mean(xv, axis=1, keepdims=True)
        var = jnp.mean(xv * xv, axis=1, keepdims=True) - mu * mu
        rstd = lax.rsqrt(var + 1e-5)
        xhat = (xv - mu) * rstd
        dgamma = jnp.sum(dyv * xhat, axis=0, keepdims=True)
        dbeta = jnp.sum(dyv, axis=0, keepdims=True)
        partial = jnp.concatenate([dgamma, dbeta], axis=0)
        send_buf[:, :] = partial

        pl.semaphore_wait(barrier_sem, N_DEV - 1)

        rdmas = []
        for dd in range(1, N_DEV):
            peer = lax.rem(my_pos + dd, N_DEV)
            rdma = pltpu.make_async_remote_copy(
                src_ref=send_buf,
                dst_ref=recv_buf.at[dd - 1],
                send_sem=send_sems.at[dd - 1],
                recv_sem=recv_sems.at[dd - 1],
                device_id=(peer,),
                device_id_type=pl.DeviceIdType.MESH,
            )
            rdma.start()
            rdmas.append(rdma)

        for rdma in rdmas:
            rdma.wait_recv()
        out_ref[:, :] = partial + jnp.sum(recv_buf[:, :, :], axis=0)

        for rdma in rdmas:
            rdma.wait_send()

    return pl.pallas_call(
        body,
        out_shape=jax.ShapeDtypeStruct((2, d), jnp.float32),
        in_specs=[
            pl.BlockSpec(memory_space=pltpu.VMEM),
            pl.BlockSpec(memory_space=pltpu.VMEM),
            pl.BlockSpec(memory_space=pltpu.VMEM),
        ],
        out_specs=pl.BlockSpec(memory_space=pltpu.VMEM),
        scratch_shapes=[
            pltpu.VMEM((2, d), jnp.float32),
            pltpu.VMEM((N_DEV - 1, 2, d), jnp.float32),
            pltpu.SemaphoreType.DMA((N_DEV - 1,)),
            pltpu.SemaphoreType.DMA((N_DEV - 1,)),
        ],
        compiler_params=pltpu.CompilerParams(collective_id=0),
    )(x, dy, gamma)
